# initial kernel scaffold (unmeasured)
import jax
import jax.numpy as jnp
from jax import lax
from jax.experimental import pallas as pl
from jax.experimental.pallas import tpu as pltpu

N_Z = 4
SEQ = 1024
HEADS = 16
DH = 128
SCALE = DH ** -0.5


def kernel(Q, K, V):
    q = jnp.transpose(Q[0], (1, 0, 2)).astype(jnp.bfloat16)
    k = jnp.transpose(K[0], (1, 0, 2)).astype(jnp.bfloat16)
    v = jnp.transpose(V[0], (1, 0, 2)).astype(jnp.bfloat16)

    def body(q_ref, k_ref, v_ref, out_ref, kf, vf, ksend, krecv, vsend, vrecv):
        my_x = lax.axis_index("x")
        my_y = lax.axis_index("y")
        my_z = lax.axis_index("z")
        right = (my_z + 1) % N_Z
        left = (my_z - 1) % N_Z

        barrier = pltpu.get_barrier_semaphore()
        for nbr in (left, right):
            pl.semaphore_signal(
                barrier, inc=1,
                device_id=(my_x, my_y, nbr),
                device_id_type=pl.DeviceIdType.MESH,
            )
        pl.semaphore_wait(barrier, 2)

        kf[my_z] = k_ref[...]
        vf[my_z] = v_ref[...]

        for h in range(N_Z - 1):
            ko = (my_z - h) % N_Z
            vo = (my_z + h) % N_Z
            krdma = pltpu.make_async_remote_copy(
                src_ref=kf.at[ko],
                dst_ref=kf.at[ko],
                send_sem=ksend.at[h],
                recv_sem=krecv.at[h],
                device_id=(my_x, my_y, right),
                device_id_type=pl.DeviceIdType.MESH,
            )
            vrdma = pltpu.make_async_remote_copy(
                src_ref=vf.at[vo],
                dst_ref=vf.at[vo],
                send_sem=vsend.at[h],
                recv_sem=vrecv.at[h],
                device_id=(my_x, my_y, left),
                device_id_type=pl.DeviceIdType.MESH,
            )
            krdma.start()
            vrdma.start()
            krdma.wait()
            vrdma.wait()

        for h in range(HEADS):
            q_h = q_ref[h]
            m = jnp.full((SEQ, 1), -jnp.inf, dtype=jnp.float32)
            l = jnp.zeros((SEQ, 1), dtype=jnp.float32)
            acc = jnp.zeros((SEQ, DH), dtype=jnp.float32)
            for o in range(N_Z):
                s = lax.dot_general(
                    q_h, kf[o, h],
                    (((1,), (1,)), ((), ())),
                    preferred_element_type=jnp.float32,
                ) * SCALE
                m_new = jnp.maximum(m, jnp.max(s, axis=-1, keepdims=True))
                p = jnp.exp(s - m_new)
                corr = jnp.exp(m - m_new)
                l = l * corr + jnp.sum(p, axis=-1, keepdims=True)
                acc = acc * corr + lax.dot_general(
                    p.astype(jnp.bfloat16), vf[o, h],
                    (((1,), (0,)), ((), ())),
                    preferred_element_type=jnp.float32,
                )
                m = m_new
            out_ref[h] = acc / l

    out = pl.pallas_call(
        body,
        out_shape=jax.ShapeDtypeStruct((HEADS, SEQ, DH), jnp.float32),
        in_specs=[pl.BlockSpec(memory_space=pltpu.VMEM)] * 3,
        out_specs=pl.BlockSpec(memory_space=pltpu.VMEM),
        scratch_shapes=[
            pltpu.VMEM((N_Z, HEADS, SEQ, DH), jnp.bfloat16),
            pltpu.VMEM((N_Z, HEADS, SEQ, DH), jnp.bfloat16),
            pltpu.SemaphoreType.DMA((N_Z - 1,)),
            pltpu.SemaphoreType.DMA((N_Z - 1,)),
            pltpu.SemaphoreType.DMA((N_Z - 1,)),
            pltpu.SemaphoreType.DMA((N_Z - 1,)),
        ],
        compiler_params=pltpu.CompilerParams(collective_id=0),
    )(q, k, v)

    return jnp.transpose(out, (1, 0, 2))[None]


# baseline (device time: 417286 ns/iter reference)
import jax
import jax.numpy as jnp
from jax import lax
from jax.experimental import pallas as pl
from jax.experimental.pallas import tpu as pltpu

N_Z = 4
SEQ = 1024
HEADS = 16
DH = 128
SCALE = DH ** -0.5


def kernel(Q, K, V):
    q = jnp.transpose(Q[0], (1, 0, 2)).astype(jnp.bfloat16)
    k = jnp.transpose(K[0], (1, 0, 2)).astype(jnp.bfloat16)
    v = jnp.transpose(V[0], (1, 0, 2)).astype(jnp.bfloat16)

    def body(q_ref, k_hbm, v_hbm, out_ref, kf, vf, ksend, krecv, vsend, vrecv,
             copy_sems):
        my_x = lax.axis_index("x")
        my_y = lax.axis_index("y")
        my_z = lax.axis_index("z")
        right = (my_z + 1) % N_Z
        left = (my_z - 1) % N_Z

        kcopy = pltpu.make_async_copy(k_hbm, kf.at[my_z], copy_sems.at[0])
        vcopy = pltpu.make_async_copy(v_hbm, vf.at[my_z], copy_sems.at[1])
        kcopy.start()
        vcopy.start()

        barrier = pltpu.get_barrier_semaphore()
        for nbr in (left, right):
            pl.semaphore_signal(
                barrier, inc=1,
                device_id=(my_x, my_y, nbr),
                device_id_type=pl.DeviceIdType.MESH,
            )
        pl.semaphore_wait(barrier, 2)
        kcopy.wait()
        vcopy.wait()

        for h in range(N_Z - 1):
            ko = (my_z - h) % N_Z
            vo = (my_z + h) % N_Z
            krdma = pltpu.make_async_remote_copy(
                src_ref=kf.at[ko],
                dst_ref=kf.at[ko],
                send_sem=ksend.at[h],
                recv_sem=krecv.at[h],
                device_id=(my_x, my_y, right),
                device_id_type=pl.DeviceIdType.MESH,
            )
            vrdma = pltpu.make_async_remote_copy(
                src_ref=vf.at[vo],
                dst_ref=vf.at[vo],
                send_sem=vsend.at[h],
                recv_sem=vrecv.at[h],
                device_id=(my_x, my_y, left),
                device_id_type=pl.DeviceIdType.MESH,
            )
            krdma.start()
            vrdma.start()
            krdma.wait()
            vrdma.wait()

        def head_body(h, carry):
            q_h = q_ref[h]
            m = jnp.full((SEQ, 1), -jnp.inf, dtype=jnp.float32)
            l = jnp.zeros((SEQ, 1), dtype=jnp.float32)
            acc = jnp.zeros((SEQ, DH), dtype=jnp.float32)
            for o in range(N_Z):
                s = lax.dot_general(
                    q_h, kf[o, h],
                    (((1,), (1,)), ((), ())),
                    preferred_element_type=jnp.float32,
                ) * SCALE
                m_new = jnp.maximum(m, jnp.max(s, axis=-1, keepdims=True))
                p = jnp.exp(s - m_new)
                corr = jnp.exp(m - m_new)
                l = l * corr + jnp.sum(p, axis=-1, keepdims=True)
                acc = acc * corr + lax.dot_general(
                    p.astype(jnp.bfloat16), vf[o, h],
                    (((1,), (0,)), ((), ())),
                    preferred_element_type=jnp.float32,
                )
                m = m_new
            out_ref[h] = (acc / l).astype(jnp.bfloat16)
            return carry

        lax.fori_loop(0, HEADS, head_body, 0)

    out = pl.pallas_call(
        body,
        out_shape=jax.ShapeDtypeStruct((HEADS, SEQ, DH), jnp.bfloat16),
        in_specs=[
            pl.BlockSpec(memory_space=pltpu.VMEM),
            pl.BlockSpec(memory_space=pl.ANY),
            pl.BlockSpec(memory_space=pl.ANY),
        ],
        out_specs=pl.BlockSpec(memory_space=pltpu.VMEM),
        scratch_shapes=[
            pltpu.VMEM((N_Z, HEADS, SEQ, DH), jnp.bfloat16),
            pltpu.VMEM((N_Z, HEADS, SEQ, DH), jnp.bfloat16),
            pltpu.SemaphoreType.DMA((N_Z - 1,)),
            pltpu.SemaphoreType.DMA((N_Z - 1,)),
            pltpu.SemaphoreType.DMA((N_Z - 1,)),
            pltpu.SemaphoreType.DMA((N_Z - 1,)),
            pltpu.SemaphoreType.DMA((2,)),
        ],
        compiler_params=pltpu.CompilerParams(
            collective_id=0,
            vmem_limit_bytes=100 * 1024 * 1024,
        ),
    )(q, k, v)

    return jnp.transpose(out, (1, 0, 2))[None].astype(jnp.float32)


# device time: 253134 ns/iter; 1.6485x vs baseline; 1.6485x over previous
import jax
import jax.numpy as jnp
from jax import lax
from jax.experimental import pallas as pl
from jax.experimental.pallas import tpu as pltpu

N_Z = 4
SEQ = 1024
HEADS = 16
HHALF = HEADS // 2
DH = 128
SCALE = DH ** -0.5


def kernel(Q, K, V):
    my_x_out = lax.axis_index("x")
    q = jnp.transpose(Q[0], (1, 0, 2)).astype(jnp.bfloat16)
    q_half = lax.dynamic_slice_in_dim(q, my_x_out * HHALF, HHALF, axis=0)
    k = jnp.transpose(K[0], (1, 0, 2)).astype(jnp.bfloat16)
    v = jnp.transpose(V[0], (1, 0, 2)).astype(jnp.bfloat16)

    def body(q_ref, k_hbm, v_hbm, out_ref, kf, vf, ksend, krecv, vsend, vrecv,
             oxsend, oxrecv, copy_sems):
        my_x = lax.axis_index("x")
        my_y = lax.axis_index("y")
        my_z = lax.axis_index("z")
        right = (my_z + 1) % N_Z
        left = (my_z - 1) % N_Z
        partner = 1 - my_x
        lo = my_x * HHALF

        kcopy = pltpu.make_async_copy(
            k_hbm.at[pl.ds(lo, HHALF)], kf.at[my_z], copy_sems.at[0])
        vcopy = pltpu.make_async_copy(
            v_hbm.at[pl.ds(lo, HHALF)], vf.at[my_z], copy_sems.at[1])
        kcopy.start()
        vcopy.start()

        barrier = pltpu.get_barrier_semaphore()
        for dev in ((my_x, my_y, left), (my_x, my_y, right),
                    (partner, my_y, my_z)):
            pl.semaphore_signal(barrier, inc=1, device_id=dev,
                                device_id_type=pl.DeviceIdType.MESH)
        pl.semaphore_wait(barrier, 3)
        kcopy.wait()
        vcopy.wait()

        for h in range(N_Z - 1):
            ko = (my_z - h) % N_Z
            vo = (my_z + h) % N_Z
            krdma = pltpu.make_async_remote_copy(
                src_ref=kf.at[ko], dst_ref=kf.at[ko],
                send_sem=ksend.at[h], recv_sem=krecv.at[h],
                device_id=(my_x, my_y, right),
                device_id_type=pl.DeviceIdType.MESH)
            vrdma = pltpu.make_async_remote_copy(
                src_ref=vf.at[vo], dst_ref=vf.at[vo],
                send_sem=vsend.at[h], recv_sem=vrecv.at[h],
                device_id=(my_x, my_y, left),
                device_id_type=pl.DeviceIdType.MESH)
            krdma.start()
            vrdma.start()
            krdma.wait()
            vrdma.wait()

        def head_body(h, carry):
            q_h = q_ref[h]
            m = jnp.full((SEQ, 1), -jnp.inf, dtype=jnp.float32)
            l = jnp.zeros((SEQ, 1), dtype=jnp.float32)
            acc = jnp.zeros((SEQ, DH), dtype=jnp.float32)
            for o in range(N_Z):
                s = lax.dot_general(
                    q_h, kf[o, h], (((1,), (1,)), ((), ())),
                    preferred_element_type=jnp.float32) * SCALE
                m_new = jnp.maximum(m, jnp.max(s, axis=-1, keepdims=True))
                p = jnp.exp(s - m_new)
                corr = jnp.exp(m - m_new)
                l = l * corr + jnp.sum(p, axis=-1, keepdims=True)
                acc = acc * corr + lax.dot_general(
                    p.astype(jnp.bfloat16), vf[o, h],
                    (((1,), (0,)), ((), ())),
                    preferred_element_type=jnp.float32)
                m = m_new
            out_ref[lo + h] = (acc / l).astype(jnp.bfloat16)
            return carry

        lax.fori_loop(0, HHALF, head_body, 0)

        oxr = pltpu.make_async_remote_copy(
            src_ref=out_ref.at[pl.ds(lo, HHALF)],
            dst_ref=out_ref.at[pl.ds(lo, HHALF)],
            send_sem=oxsend, recv_sem=oxrecv,
            device_id=(partner, my_y, my_z),
            device_id_type=pl.DeviceIdType.MESH)
        oxr.start()
        oxr.wait()

    out = pl.pallas_call(
        body,
        out_shape=jax.ShapeDtypeStruct((HEADS, SEQ, DH), jnp.bfloat16),
        in_specs=[
            pl.BlockSpec(memory_space=pltpu.VMEM),
            pl.BlockSpec(memory_space=pl.ANY),
            pl.BlockSpec(memory_space=pl.ANY),
        ],
        out_specs=pl.BlockSpec(memory_space=pltpu.VMEM),
        scratch_shapes=[
            pltpu.VMEM((N_Z, HHALF, SEQ, DH), jnp.bfloat16),
            pltpu.VMEM((N_Z, HHALF, SEQ, DH), jnp.bfloat16),
            pltpu.SemaphoreType.DMA((N_Z - 1,)),
            pltpu.SemaphoreType.DMA((N_Z - 1,)),
            pltpu.SemaphoreType.DMA((N_Z - 1,)),
            pltpu.SemaphoreType.DMA((N_Z - 1,)),
            pltpu.SemaphoreType.DMA,
            pltpu.SemaphoreType.DMA,
            pltpu.SemaphoreType.DMA((2,)),
        ],
        compiler_params=pltpu.CompilerParams(
            collective_id=0,
            vmem_limit_bytes=100 * 1024 * 1024,
        ),
    )(q_half, k, v)

    return jnp.transpose(out, (1, 0, 2))[None].astype(jnp.float32)


# device time: 214353 ns/iter; 1.9467x vs baseline; 1.1809x over previous
import jax
import jax.numpy as jnp
from jax import lax
from jax.experimental import pallas as pl
from jax.experimental.pallas import tpu as pltpu

N_Z = 4
SEQ = 1024
HEADS = 16
HHALF = HEADS // 2
HQ = HHALF // 2
DH = 128
SCALE = DH ** -0.5


def kernel(Q, K, V):
    my_x_out = lax.axis_index("x")
    q = jnp.transpose(Q[0], (1, 0, 2)).astype(jnp.bfloat16)
    q_half = lax.dynamic_slice_in_dim(q, my_x_out * HHALF, HHALF, axis=0)
    k = jnp.transpose(K[0], (1, 0, 2)).astype(jnp.bfloat16)
    v = jnp.transpose(V[0], (1, 0, 2)).astype(jnp.bfloat16)

    def body(q_ref, k_hbm, v_hbm, out_ref, kf, vf, acc, lsum,
             send_sems, recv_sems, oxsend, oxrecv, copy_sems):
        my_x = lax.axis_index("x")
        my_y = lax.axis_index("y")
        my_z = lax.axis_index("z")
        right = (my_z + 1) % N_Z
        left = (my_z - 1) % N_Z
        partner = 1 - my_x
        lo = my_x * HHALF

        kcopy = pltpu.make_async_copy(
            k_hbm.at[pl.ds(lo, HHALF)], kf.at[my_z], copy_sems.at[0])
        vcopy = pltpu.make_async_copy(
            v_hbm.at[pl.ds(lo, HHALF)], vf.at[my_z], copy_sems.at[1])
        kcopy.start()
        vcopy.start()

        barrier = pltpu.get_barrier_semaphore()
        for dev in ((my_x, my_y, left), (my_x, my_y, right),
                    (partner, my_y, my_z)):
            pl.semaphore_signal(barrier, inc=1, device_id=dev,
                                device_id_type=pl.DeviceIdType.MESH)
        pl.semaphore_wait(barrier, 3)
        kcopy.wait()
        vcopy.wait()

        def accumulate(t):
            o_cw = (my_z - t) % N_Z
            o_ccw = (my_z + t) % N_Z

            def head_body(h, carry):
                o = jnp.where(h < HQ, o_cw, o_ccw)
                s = lax.dot_general(
                    q_ref[h], kf[o, h], (((1,), (1,)), ((), ())),
                    preferred_element_type=jnp.float32) * SCALE
                e = jnp.exp(s)
                lsum[h] += jnp.broadcast_to(
                    jnp.sum(e, axis=-1, keepdims=True), (SEQ, DH))
                acc[h] += lax.dot_general(
                    e.astype(jnp.bfloat16), vf[o, h],
                    (((1,), (0,)), ((), ())),
                    preferred_element_type=jnp.float32)
                return carry

            lax.fori_loop(0, HHALF, head_body, 0)

        acc[...] = jnp.zeros((HHALF, SEQ, DH), jnp.float32)
        lsum[...] = jnp.zeros((HHALF, SEQ, DH), jnp.float32)

        for h in range(N_Z - 1):
            o_cw = (my_z - h) % N_Z
            o_ccw = (my_z + h) % N_Z
            rdmas = []
            for stream, (buf, o, hlo, dst) in enumerate((
                    (kf, o_cw, 0, right),
                    (vf, o_cw, 0, right),
                    (kf, o_ccw, HQ, left),
                    (vf, o_ccw, HQ, left))):
                r = pltpu.make_async_remote_copy(
                    src_ref=buf.at[o, pl.ds(hlo, HQ)],
                    dst_ref=buf.at[o, pl.ds(hlo, HQ)],
                    send_sem=send_sems.at[stream, h],
                    recv_sem=recv_sems.at[stream, h],
                    device_id=(my_x, my_y, dst),
                    device_id_type=pl.DeviceIdType.MESH)
                r.start()
                rdmas.append(r)
            accumulate(h)
            for r in rdmas:
                r.wait()
        accumulate(N_Z - 1)

        def finish_body(h, carry):
            out_ref[lo + h] = (acc[h] / lsum[h]).astype(jnp.bfloat16)
            return carry

        lax.fori_loop(0, HHALF, finish_body, 0)

        oxr = pltpu.make_async_remote_copy(
            src_ref=out_ref.at[pl.ds(lo, HHALF)],
            dst_ref=out_ref.at[pl.ds(lo, HHALF)],
            send_sem=oxsend, recv_sem=oxrecv,
            device_id=(partner, my_y, my_z),
            device_id_type=pl.DeviceIdType.MESH)
        oxr.start()
        oxr.wait()

    out = pl.pallas_call(
        body,
        out_shape=jax.ShapeDtypeStruct((HEADS, SEQ, DH), jnp.bfloat16),
        in_specs=[
            pl.BlockSpec(memory_space=pltpu.VMEM),
            pl.BlockSpec(memory_space=pl.ANY),
            pl.BlockSpec(memory_space=pl.ANY),
        ],
        out_specs=pl.BlockSpec(memory_space=pltpu.VMEM),
        scratch_shapes=[
            pltpu.VMEM((N_Z, HHALF, SEQ, DH), jnp.bfloat16),
            pltpu.VMEM((N_Z, HHALF, SEQ, DH), jnp.bfloat16),
            pltpu.VMEM((HHALF, SEQ, DH), jnp.float32),
            pltpu.VMEM((HHALF, SEQ, DH), jnp.float32),
            pltpu.SemaphoreType.DMA((4, N_Z - 1)),
            pltpu.SemaphoreType.DMA((4, N_Z - 1)),
            pltpu.SemaphoreType.DMA,
            pltpu.SemaphoreType.DMA,
            pltpu.SemaphoreType.DMA((2,)),
        ],
        compiler_params=pltpu.CompilerParams(
            collective_id=0,
            vmem_limit_bytes=100 * 1024 * 1024,
        ),
    )(q_half, k, v)

    return jnp.transpose(out, (1, 0, 2))[None].astype(jnp.float32)


# device time: 124268 ns/iter; 3.3580x vs baseline; 1.7249x over previous
import jax
import jax.numpy as jnp
from jax import lax
from jax.experimental import pallas as pl
from jax.experimental.pallas import tpu as pltpu

N_Z = 4
N_Y = 4
SEQ = 1024
HEADS = 16
HMINE = 2
DH = 128
SCALE = DH ** -0.5


def kernel(Q, K, V):
    x_o = lax.axis_index("x")
    y_o = lax.axis_index("y")
    lo_o = 4 * y_o + 2 * x_o
    q = jnp.transpose(Q[0], (1, 0, 2)).astype(jnp.bfloat16)
    q_mine = lax.dynamic_slice_in_dim(q, lo_o, HMINE, axis=0)
    k = jnp.transpose(K[0], (1, 0, 2)).astype(jnp.bfloat16)
    v = jnp.transpose(V[0], (1, 0, 2)).astype(jnp.bfloat16)

    def body(q_ref, k_hbm, v_hbm, out_ref, kf, vf, acc, lsum,
             zsend, zrecv, oxsend, oxrecv, ysend, yrecv, copy_sems):
        my_x = lax.axis_index("x")
        my_y = lax.axis_index("y")
        my_z = lax.axis_index("z")
        z_r = (my_z + 1) % N_Z
        z_l = (my_z - 1) % N_Z
        y_u = (my_y + 1) % N_Y
        y_d = (my_y - 1) % N_Y
        partner = 1 - my_x
        lo = 4 * my_y + 2 * my_x

        kcopy = pltpu.make_async_copy(
            k_hbm.at[pl.ds(lo, HMINE)], kf.at[my_z], copy_sems.at[0])
        vcopy = pltpu.make_async_copy(
            v_hbm.at[pl.ds(lo, HMINE)], vf.at[my_z], copy_sems.at[1])
        kcopy.start()
        vcopy.start()

        barrier = pltpu.get_barrier_semaphore()
        for dev in ((my_x, my_y, z_l), (my_x, my_y, z_r),
                    (partner, my_y, my_z),
                    (my_x, y_d, my_z), (my_x, y_u, my_z)):
            pl.semaphore_signal(barrier, inc=1, device_id=dev,
                                device_id_type=pl.DeviceIdType.MESH)
        pl.semaphore_wait(barrier, 5)
        kcopy.wait()
        vcopy.wait()

        acc[...] = jnp.zeros((HMINE, SEQ, DH), jnp.float32)
        lsum[...] = jnp.zeros((HMINE, SEQ, DH), jnp.float32)

        def accumulate(t):
            for i, o in ((0, (my_z - t) % N_Z), (1, (my_z + t) % N_Z)):
                s = lax.dot_general(
                    q_ref[i], kf[o, i], (((1,), (1,)), ((), ())),
                    preferred_element_type=jnp.float32) * SCALE
                e = jnp.exp(s)
                lsum[i] += jnp.broadcast_to(
                    jnp.sum(e, axis=-1, keepdims=True), (SEQ, DH))
                acc[i] += lax.dot_general(
                    e.astype(jnp.bfloat16), vf[o, i],
                    (((1,), (0,)), ((), ())),
                    preferred_element_type=jnp.float32)

        for h in range(N_Z - 1):
            o_cw = (my_z - h) % N_Z
            o_ccw = (my_z + h) % N_Z
            rdmas = []
            for stream, (buf, o, hi, dst) in enumerate((
                    (kf, o_cw, 0, z_r),
                    (vf, o_cw, 0, z_r),
                    (kf, o_ccw, 1, z_l),
                    (vf, o_ccw, 1, z_l))):
                r = pltpu.make_async_remote_copy(
                    src_ref=buf.at[o, pl.ds(hi, 1)],
                    dst_ref=buf.at[o, pl.ds(hi, 1)],
                    send_sem=zsend.at[stream, h],
                    recv_sem=zrecv.at[stream, h],
                    device_id=(my_x, my_y, dst),
                    device_id_type=pl.DeviceIdType.MESH)
                r.start()
                rdmas.append(r)
            accumulate(h)
            for r in rdmas:
                r.wait()
        accumulate(N_Z - 1)

        for i in range(HMINE):
            out_ref[lo + i] = (acc[i] / lsum[i]).astype(jnp.bfloat16)

        oxr = pltpu.make_async_remote_copy(
            src_ref=out_ref.at[pl.ds(lo, HMINE)],
            dst_ref=out_ref.at[pl.ds(lo, HMINE)],
            send_sem=oxsend, recv_sem=oxrecv,
            device_id=(partner, my_y, my_z),
            device_id_type=pl.DeviceIdType.MESH)
        oxr.start()
        oxr.wait()

        for h in range(N_Y - 1):
            b_cw = (my_y - h) % N_Y
            b_ccw = (my_y + h) % N_Y
            rdmas = []
            for stream, (blk, off, dst) in enumerate((
                    (b_cw, 0, y_u), (b_ccw, 2, y_d))):
                r = pltpu.make_async_remote_copy(
                    src_ref=out_ref.at[pl.ds(4 * blk + off, 2)],
                    dst_ref=out_ref.at[pl.ds(4 * blk + off, 2)],
                    send_sem=ysend.at[stream, h],
                    recv_sem=yrecv.at[stream, h],
                    device_id=(my_x, dst, my_z),
                    device_id_type=pl.DeviceIdType.MESH)
                r.start()
                rdmas.append(r)
            for r in rdmas:
                r.wait()

    out = pl.pallas_call(
        body,
        out_shape=jax.ShapeDtypeStruct((HEADS, SEQ, DH), jnp.bfloat16),
        in_specs=[
            pl.BlockSpec(memory_space=pltpu.VMEM),
            pl.BlockSpec(memory_space=pl.ANY),
            pl.BlockSpec(memory_space=pl.ANY),
        ],
        out_specs=pl.BlockSpec(memory_space=pltpu.VMEM),
        scratch_shapes=[
            pltpu.VMEM((N_Z, HMINE, SEQ, DH), jnp.bfloat16),
            pltpu.VMEM((N_Z, HMINE, SEQ, DH), jnp.bfloat16),
            pltpu.VMEM((HMINE, SEQ, DH), jnp.float32),
            pltpu.VMEM((HMINE, SEQ, DH), jnp.float32),
            pltpu.SemaphoreType.DMA((4, N_Z - 1)),
            pltpu.SemaphoreType.DMA((4, N_Z - 1)),
            pltpu.SemaphoreType.DMA,
            pltpu.SemaphoreType.DMA,
            pltpu.SemaphoreType.DMA((2, N_Y - 1)),
            pltpu.SemaphoreType.DMA((2, N_Y - 1)),
            pltpu.SemaphoreType.DMA((2,)),
        ],
        compiler_params=pltpu.CompilerParams(
            collective_id=0,
            vmem_limit_bytes=100 * 1024 * 1024,
        ),
    )(q_mine, k, v)

    return jnp.transpose(out, (1, 0, 2))[None].astype(jnp.float32)


# device time: 115871 ns/iter; 3.6013x vs baseline; 1.0725x over previous
import jax
import jax.numpy as jnp
from jax import lax
from jax.experimental import pallas as pl
from jax.experimental.pallas import tpu as pltpu

N_Z = 4
N_Y = 4
SEQ = 1024
HEADS = 16
HMINE = 2
DH = 128
SCALE = DH ** -0.5


def kernel(Q, K, V):
    def body(q_hbm, k_hbm, v_hbm, out_ref, kf, vf, qloc, acc, lsum, outg,
             qt, kt, vt, zsend, zrecv, oxsend, oxrecv, ysend, yrecv,
             copy_sems):
        my_x = lax.axis_index("x")
        my_y = lax.axis_index("y")
        my_z = lax.axis_index("z")
        z_r = (my_z + 1) % N_Z
        z_l = (my_z - 1) % N_Z
        y_u = (my_y + 1) % N_Y
        y_d = (my_y - 1) % N_Y
        partner = 1 - my_x
        lo = 4 * my_y + 2 * my_x

        copies = [
            pltpu.make_async_copy(
                src.at[0, :, pl.ds(lo, HMINE), :], dst, copy_sems.at[i])
            for i, (src, dst) in enumerate(
                ((q_hbm, qt), (k_hbm, kt), (v_hbm, vt)))
        ]
        for c in copies:
            c.start()

        barrier = pltpu.get_barrier_semaphore()
        for dev in ((my_x, my_y, z_l), (my_x, my_y, z_r),
                    (partner, my_y, my_z),
                    (my_x, y_d, my_z), (my_x, y_u, my_z)):
            pl.semaphore_signal(barrier, inc=1, device_id=dev,
                                device_id_type=pl.DeviceIdType.MESH)
        pl.semaphore_wait(barrier, 5)
        for c in copies:
            c.wait()

        for i in range(HMINE):
            qloc[i] = qt[:, i, :].astype(jnp.bfloat16)
            kf[my_z, i] = kt[:, i, :].astype(jnp.bfloat16)
            vf[my_z, i] = vt[:, i, :].astype(jnp.bfloat16)

        acc[...] = jnp.zeros((HMINE, SEQ, DH), jnp.float32)
        lsum[...] = jnp.zeros((HMINE, SEQ, DH), jnp.float32)

        def accumulate(t):
            for i, o in ((0, (my_z - t) % N_Z), (1, (my_z + t) % N_Z)):
                s = lax.dot_general(
                    qloc[i], kf[o, i], (((1,), (1,)), ((), ())),
                    preferred_element_type=jnp.float32) * SCALE
                e = jnp.exp(s)
                lsum[i] += jnp.broadcast_to(
                    jnp.sum(e, axis=-1, keepdims=True), (SEQ, DH))
                acc[i] += lax.dot_general(
                    e.astype(jnp.bfloat16), vf[o, i],
                    (((1,), (0,)), ((), ())),
                    preferred_element_type=jnp.float32)

        for h in range(N_Z - 1):
            o_cw = (my_z - h) % N_Z
            o_ccw = (my_z + h) % N_Z
            rdmas = []
            for stream, (buf, o, hi, dst) in enumerate((
                    (kf, o_cw, 0, z_r),
                    (vf, o_cw, 0, z_r),
                    (kf, o_ccw, 1, z_l),
                    (vf, o_ccw, 1, z_l))):
                r = pltpu.make_async_remote_copy(
                    src_ref=buf.at[o, pl.ds(hi, 1)],
                    dst_ref=buf.at[o, pl.ds(hi, 1)],
                    send_sem=zsend.at[stream, h],
                    recv_sem=zrecv.at[stream, h],
                    device_id=(my_x, my_y, dst),
                    device_id_type=pl.DeviceIdType.MESH)
                r.start()
                rdmas.append(r)
            accumulate(h)
            for r in rdmas:
                r.wait()
        accumulate(N_Z - 1)

        for i in range(HMINE):
            outg[lo + i] = (acc[i] / lsum[i]).astype(jnp.bfloat16)

        oxr = pltpu.make_async_remote_copy(
            src_ref=outg.at[pl.ds(lo, HMINE)],
            dst_ref=outg.at[pl.ds(lo, HMINE)],
            send_sem=oxsend, recv_sem=oxrecv,
            device_id=(partner, my_y, my_z),
            device_id_type=pl.DeviceIdType.MESH)
        oxr.start()
        oxr.wait()

        for h in range(N_Y - 1):
            b_cw = (my_y - h) % N_Y
            b_ccw = (my_y + h) % N_Y
            rdmas = []
            for stream, (blk, off, dst) in enumerate((
                    (b_cw, 0, y_u), (b_ccw, 2, y_d))):
                r = pltpu.make_async_remote_copy(
                    src_ref=outg.at[pl.ds(4 * blk + off, 2)],
                    dst_ref=outg.at[pl.ds(4 * blk + off, 2)],
                    send_sem=ysend.at[stream, h],
                    recv_sem=yrecv.at[stream, h],
                    device_id=(my_x, dst, my_z),
                    device_id_type=pl.DeviceIdType.MESH)
                r.start()
                rdmas.append(r)
            for r in rdmas:
                r.wait()

        for i in range(HEADS):
            out_ref[:, i, :] = outg[i].astype(jnp.float32)

    out = pl.pallas_call(
        body,
        out_shape=jax.ShapeDtypeStruct((SEQ, HEADS, DH), jnp.float32),
        in_specs=[pl.BlockSpec(memory_space=pl.ANY)] * 3,
        out_specs=pl.BlockSpec(memory_space=pltpu.VMEM),
        scratch_shapes=[
            pltpu.VMEM((N_Z, HMINE, SEQ, DH), jnp.bfloat16),
            pltpu.VMEM((N_Z, HMINE, SEQ, DH), jnp.bfloat16),
            pltpu.VMEM((HMINE, SEQ, DH), jnp.bfloat16),
            pltpu.VMEM((HMINE, SEQ, DH), jnp.float32),
            pltpu.VMEM((HMINE, SEQ, DH), jnp.float32),
            pltpu.VMEM((HEADS, SEQ, DH), jnp.bfloat16),
            pltpu.VMEM((SEQ, HMINE, DH), jnp.float32),
            pltpu.VMEM((SEQ, HMINE, DH), jnp.float32),
            pltpu.VMEM((SEQ, HMINE, DH), jnp.float32),
            pltpu.SemaphoreType.DMA((4, N_Z - 1)),
            pltpu.SemaphoreType.DMA((4, N_Z - 1)),
            pltpu.SemaphoreType.DMA,
            pltpu.SemaphoreType.DMA,
            pltpu.SemaphoreType.DMA((2, N_Y - 1)),
            pltpu.SemaphoreType.DMA((2, N_Y - 1)),
            pltpu.SemaphoreType.DMA((3,)),
        ],
        compiler_params=pltpu.CompilerParams(
            collective_id=0,
            vmem_limit_bytes=100 * 1024 * 1024,
        ),
    )(Q, K, V)

    return out[None]
